# baseline (speedup 1.0000x reference)
import functools

import jax
import jax.numpy as jnp
from jax import lax
from jax.experimental import pallas as pl
from jax.experimental.pallas import tpu as pltpu
from jax.experimental.pallas import tpu_sc as plsc

NW = 32
L = 16
NB = 4


def _wid():
    return lax.axis_index("s") * 2 + lax.axis_index("c")


def _face_table_body(RO, GS, verts_hbm, facesT_hbm, fcT_hbm, idx_v, vals_v, sem):
    wid = _wid()
    NG = RO // GS

    def fire(g):
        for t in range(GS):
            pltpu.make_async_copy(
                verts_hbm.at[idx_v.at[g * GS + t]],
                vals_v.at[g * GS + t], sem).start()

    def drain(g, k):
        pltpu.make_async_copy(
            fcT_hbm.at[k, wid, pl.ds(0, GS)],
            vals_v.at[pl.ds(g * GS, GS)], sem).wait()

    for k in range(3):
        pltpu.sync_copy(facesT_hbm.at[k, wid], idx_v)
        fire(0)

        def step(g, carry):
            fire(g + 1)
            drain(g, k)
            return carry

        lax.fori_loop(0, NG - 1, step, 0)
        drain(NG - 1, k)
        pltpu.sync_copy(vals_v, fcT_hbm.at[k, wid])


def _shade_body(PT, CH, *refs):
    fcs = refs[0:3]
    p2f_hbm = refs[3]
    bhs = refs[4:7]
    out_hbm = refs[7]
    rest = refs[8:]
    idxs = rest[0:NB]
    bss = [rest[NB + q * 3: NB + q * 3 + 3] for q in range(NB)]
    gss = [rest[NB + 3 * NB + q * 3: NB + 3 * NB + q * 3 + 3] for q in range(NB)]
    outs = rest[NB + 6 * NB: NB + 6 * NB + NB]
    sins = rest[NB + 7 * NB: NB + 7 * NB + NB]
    sgs = rest[NB + 8 * NB: NB + 8 * NB + NB]
    sos = rest[NB + 9 * NB: NB + 9 * NB + NB]

    wid = _wid()
    NR = CH // 128
    NC = PT // CH

    def in_descs(c, q):
        row0 = wid * (PT // 128) + c * NR
        base = wid * PT + c * CH
        ds = [pltpu.make_async_copy(
            p2f_hbm.at[pl.ds(row0, NR)], idxs[q], sins[q])]
        for k in range(3):
            ds.append(pltpu.make_async_copy(
                bhs[k].at[pl.ds(base, CH)], bss[q][k], sins[q]))
        return ds

    def start_in(c, q):
        for d in in_descs(c, q):
            d.start()

    def wait_in(c, q):
        for d in in_descs(c, q):
            d.wait()

    def fire_g(q):
        for k in range(3):
            for j in range(NR):
                pltpu.make_async_copy(
                    fcs[k].at[idxs[q].at[j]],
                    gss[q][k].at[pl.ds(j * 128, 128)], sgs[q]).start()

    def drain_g(q):
        for k in range(3):
            pltpu.make_async_copy(
                fcs[k].at[pl.ds(0, CH)], gss[q][k], sgs[q]).wait()

    def out_desc(c, q):
        base = wid * PT + c * CH
        return pltpu.make_async_copy(
            outs[q], out_hbm.at[pl.ds(base, CH)], sos[q])

    def compute(q):
        gs, bs, out_v = gss[q], bss[q], outs[q]

        def pix(i, carry2):
            for u in range(4):
                s = pl.ds(i * 4 * L + u * L, L)
                acc = gs[0][s] * bs[0][s]
                acc = acc + gs[1][s] * bs[1][s]
                acc = acc + gs[2][s] * bs[2][s]
                out_v[s] = acc.astype(jnp.int32)
            return carry2

        lax.fori_loop(0, CH // L // 4, pix, 0)

    def step(c, q, first):
        drain_g(q)
        wait_in(c + 2, (q + 2) % NB)
        fire_g((q + 2) % NB)
        if not first:
            out_desc(c - NB, q).wait()
        compute(q)
        out_desc(c, q).start()
        start_in(c + 3, (q + 3) % NB)

    start_in(0, 0)
    start_in(1, 1)
    wait_in(0, 0)
    fire_g(0)
    wait_in(1, 1)
    fire_g(1)
    start_in(2, 2)

    for c in range(NB):
        step(c, c % NB, True)

    def loop_body(t, carry):
        c0 = NB + t * NB
        for u in range(NB):
            step(c0 + u, u, False)
        return carry

    lax.fori_loop(0, (NC - 2 * NB) // NB, loop_body, 0)

    for c in range(NC - NB, NC):
        q = c % NB
        drain_g(q)
        if c + 2 < NC:
            wait_in(c + 2, (c + 2) % NB)
            fire_g((c + 2) % NB)
        out_desc(c - NB, q).wait()
        compute(q)
        out_desc(c, q).start()
        if c + 3 < NC:
            start_in(c + 3, (c + 3) % NB)

    for c in range(NC - NB, NC):
        out_desc(c, c % NB).wait()


def kernel(verts_features, faces, pix_to_face, bary_coords):
    N, V, C = verts_features.shape
    F = faces.shape[0]
    _, H, W, K = pix_to_face.shape
    P = N * H * W * K

    mesh = plsc.VectorSubcoreMesh(core_axis_name="c", subcore_axis_name="s")

    RO = -(-F // (NW * 128))
    GS = 7 if RO % 7 == 0 else 1
    Fp = NW * RO * 128
    verts_flat = verts_features.reshape(N * V)
    facesT = faces.T
    facesT = jnp.pad(facesT, ((0, 0), (0, Fp - F)))
    facesT = facesT.reshape(3, NW, RO, 128)

    face_table = pl.kernel(
        functools.partial(_face_table_body, RO, GS),
        mesh=mesh,
        out_type=jax.ShapeDtypeStruct((3, NW, RO, 128), jnp.float32),
        scratch_types=[
            pltpu.VMEM((RO, 128), jnp.int32),
            pltpu.VMEM((RO, 128), jnp.float32),
            pltpu.SemaphoreType.DMA,
        ],
    )
    fcT = face_table(verts_flat, facesT).reshape(3, Fp)

    PT = P // NW
    CH = 1024
    p2f = pix_to_face.reshape(P // 128, 128)
    baryT = bary_coords.reshape(P, 3).T

    shade = pl.kernel(
        functools.partial(_shade_body, PT, CH),
        mesh=mesh,
        out_type=jax.ShapeDtypeStruct((P,), jnp.int32),
        scratch_types=(
            [pltpu.VMEM((CH // 128, 128), jnp.int32)] * NB
            + [pltpu.VMEM((CH,), jnp.float32)] * (3 * NB)
            + [pltpu.VMEM((CH,), jnp.float32)] * (3 * NB)
            + [pltpu.VMEM((CH,), jnp.int32)] * NB
            + [pltpu.SemaphoreType.DMA] * (3 * NB)
        ),
    )
    out = shade(fcT[0], fcT[1], fcT[2], p2f, baryT[0], baryT[1], baryT[2])
    return out.reshape(N, H, W, K)


# speedup vs baseline: 1.5585x; 1.5585x over previous
import functools

import jax
import jax.numpy as jnp
from jax import lax
from jax.experimental import pallas as pl
from jax.experimental.pallas import tpu as pltpu
from jax.experimental.pallas import tpu_sc as plsc

NW = 32
L = 16
NB = 4


def _wid():
    return lax.axis_index("s") * 2 + lax.axis_index("c")


def _face_table_body(RO, GS, verts_hbm, facesT_hbm, fcT_hbm, idx_v, vals_v, sem):
    wid = _wid()
    NG = RO // GS

    def fire(g):
        for t in range(GS):
            pltpu.make_async_copy(
                verts_hbm.at[idx_v.at[g * GS + t]],
                vals_v.at[g * GS + t], sem).start()

    def drain(g, k):
        pltpu.make_async_copy(
            fcT_hbm.at[k, wid, pl.ds(0, GS)],
            vals_v.at[pl.ds(g * GS, GS)], sem).wait()

    for k in range(3):
        pltpu.sync_copy(facesT_hbm.at[k, wid], idx_v)
        fire(0)

        def step(g, carry):
            fire(g + 1)
            drain(g, k)
            return carry

        lax.fori_loop(0, NG - 1, step, 0)
        drain(NG - 1, k)
        pltpu.sync_copy(vals_v, fcT_hbm.at[k, wid])


def _shade_body(PT, CH, *refs):
    fcs = refs[0:3]
    p2f_hbm = refs[3]
    bhs = refs[4:7]
    out_hbm = refs[7]
    rest = refs[8:]
    shs = rest[0:3]
    rest = rest[3:]
    idxs = rest[0:NB]
    bss = [rest[NB + q * 3: NB + q * 3 + 3] for q in range(NB)]
    gss = [rest[NB + 3 * NB + q * 3: NB + 3 * NB + q * 3 + 3] for q in range(NB)]
    outs = rest[NB + 6 * NB: NB + 6 * NB + NB]
    sins = rest[NB + 7 * NB: NB + 7 * NB + NB]
    sgs = rest[NB + 8 * NB: NB + 8 * NB + NB]
    sos = rest[NB + 9 * NB: NB + 9 * NB + NB]

    wid = _wid()
    NR = CH // 128
    NC = PT // CH

    def in_descs(c, q):
        row0 = wid * (PT // 128) + c * NR
        base = wid * PT + c * CH
        ds = [pltpu.make_async_copy(
            p2f_hbm.at[pl.ds(row0, NR)], idxs[q], sins[q])]
        for k in range(3):
            ds.append(pltpu.make_async_copy(
                bhs[k].at[pl.ds(base, CH)], bss[q][k], sins[q]))
        return ds

    def start_in(c, q):
        for d in in_descs(c, q):
            d.start()

    def wait_in(c, q):
        for d in in_descs(c, q):
            d.wait()

    def fire_g(q):
        for k in range(3):
            for j in range(NR):
                pltpu.make_async_copy(
                    shs[k].at[idxs[q].at[j]],
                    gss[q][k].at[pl.ds(j * 128, 128)], sgs[q]).start()

    def drain_g(q):
        for k in range(3):
            pltpu.make_async_copy(
                fcs[k].at[pl.ds(0, CH)], gss[q][k], sgs[q]).wait()

    def out_desc(c, q):
        base = wid * PT + c * CH
        return pltpu.make_async_copy(
            outs[q], out_hbm.at[pl.ds(base, CH)], sos[q])

    def compute(q):
        gs, bs, out_v = gss[q], bss[q], outs[q]

        def pix(i, carry2):
            for u in range(4):
                s = pl.ds(i * 4 * L + u * L, L)
                acc = gs[0][s] * bs[0][s]
                acc = acc + gs[1][s] * bs[1][s]
                acc = acc + gs[2][s] * bs[2][s]
                out_v[s] = acc.astype(jnp.int32)
            return carry2

        lax.fori_loop(0, CH // L // 4, pix, 0)

    def step(c, q, first):
        drain_g(q)
        wait_in(c + 2, (q + 2) % NB)
        fire_g((q + 2) % NB)
        if not first:
            out_desc(c - NB, q).wait()
        compute(q)
        out_desc(c, q).start()
        start_in(c + 3, (q + 3) % NB)

    sid = lax.axis_index("s")
    Fp = fcs[0].shape[0]
    SL = Fp // 16
    for k in range(3):
        stage = pltpu.make_async_copy(
            fcs[k].at[pl.ds(sid * SL, SL)],
            shs[k].at[pl.ds(sid * SL, SL)], sgs[0])
        stage.start()
        stage.wait()
    plsc.subcore_barrier()

    start_in(0, 0)
    start_in(1, 1)
    wait_in(0, 0)
    fire_g(0)
    wait_in(1, 1)
    fire_g(1)
    start_in(2, 2)

    for c in range(NB):
        step(c, c % NB, True)

    def loop_body(t, carry):
        c0 = NB + t * NB
        for u in range(NB):
            step(c0 + u, u, False)
        return carry

    lax.fori_loop(0, (NC - 2 * NB) // NB, loop_body, 0)

    for c in range(NC - NB, NC):
        q = c % NB
        drain_g(q)
        if c + 2 < NC:
            wait_in(c + 2, (c + 2) % NB)
            fire_g((c + 2) % NB)
        out_desc(c - NB, q).wait()
        compute(q)
        out_desc(c, q).start()
        if c + 3 < NC:
            start_in(c + 3, (c + 3) % NB)

    for c in range(NC - NB, NC):
        out_desc(c, c % NB).wait()


def kernel(verts_features, faces, pix_to_face, bary_coords):
    N, V, C = verts_features.shape
    F = faces.shape[0]
    _, H, W, K = pix_to_face.shape
    P = N * H * W * K

    mesh = plsc.VectorSubcoreMesh(core_axis_name="c", subcore_axis_name="s")

    RO = -(-F // (NW * 128))
    GS = 7 if RO % 7 == 0 else 1
    Fp = NW * RO * 128
    verts_flat = verts_features.reshape(N * V)
    facesT = faces.T
    facesT = jnp.pad(facesT, ((0, 0), (0, Fp - F)))
    facesT = facesT.reshape(3, NW, RO, 128)

    face_table = pl.kernel(
        functools.partial(_face_table_body, RO, GS),
        mesh=mesh,
        out_type=jax.ShapeDtypeStruct((3, NW, RO, 128), jnp.float32),
        scratch_types=[
            pltpu.VMEM((RO, 128), jnp.int32),
            pltpu.VMEM((RO, 128), jnp.float32),
            pltpu.SemaphoreType.DMA,
        ],
    )
    fcT = face_table(verts_flat, facesT).reshape(3, Fp)

    PT = P // NW
    CH = 1024
    p2f = pix_to_face.reshape(P // 128, 128)
    baryT = bary_coords.reshape(P, 3).T

    shade = pl.kernel(
        functools.partial(_shade_body, PT, CH),
        mesh=mesh,
        out_type=jax.ShapeDtypeStruct((P,), jnp.int32),
        scratch_types=(
            [pltpu.VMEM_SHARED((Fp,), jnp.float32)] * 3
            + [pltpu.VMEM((CH // 128, 128), jnp.int32)] * NB
            + [pltpu.VMEM((CH,), jnp.float32)] * (3 * NB)
            + [pltpu.VMEM((CH,), jnp.float32)] * (3 * NB)
            + [pltpu.VMEM((CH,), jnp.int32)] * NB
            + [pltpu.SemaphoreType.DMA] * (3 * NB)
        ),
    )
    out = shade(fcT[0], fcT[1], fcT[2], p2f, baryT[0], baryT[1], baryT[2])
    return out.reshape(N, H, W, K)


# speedup vs baseline: 1.7659x; 1.1331x over previous
import functools

import jax
import jax.numpy as jnp
from jax import lax
from jax.experimental import pallas as pl
from jax.experimental.pallas import tpu as pltpu
from jax.experimental.pallas import tpu_sc as plsc

NW = 32
NS = 16
L = 16
NB = 4
GS = 7


def _wid():
    return lax.axis_index("s") * 2 + lax.axis_index("c")


def _shade_body(PT, CH, NVp, Fp, *refs):
    verts_hbm, facesT_hbm, p2f_hbm = refs[0:3]
    bhs = refs[3:6]
    out_hbm = refs[6]
    rest = refs[7:]
    shv = rest[0]
    shs = rest[1:4]
    fidx_v, vals_v = rest[4:6]
    rest = rest[6:]
    idxs = rest[0:NB]
    bss = [rest[NB + q * 3: NB + q * 3 + 3] for q in range(NB)]
    gss = [rest[NB + 3 * NB + q * 3: NB + 3 * NB + q * 3 + 3] for q in range(NB)]
    outs = rest[NB + 6 * NB: NB + 6 * NB + NB]
    sins = rest[NB + 7 * NB: NB + 7 * NB + NB]
    sgs = rest[NB + 8 * NB: NB + 8 * NB + NB]
    sos = rest[NB + 9 * NB: NB + 9 * NB + NB]

    wid = _wid()
    sid = lax.axis_index("s")
    NR = CH // 128
    NC = PT // CH

    VSL = NVp // NS
    SL = Fp // NS
    RO = SL // 128
    NG = RO // GS

    vstage = pltpu.make_async_copy(
        verts_hbm.at[pl.ds(sid * VSL, VSL)],
        shv.at[pl.ds(sid * VSL, VSL)], sgs[0])
    vstage.start()
    vstage.wait()
    plsc.subcore_barrier()

    def fire_a(g):
        for t in range(GS):
            r = g * GS + t
            pltpu.make_async_copy(
                shv.at[fidx_v.at[r]],
                vals_v.at[pl.ds(r * 128, 128)], sgs[0]).start()

    def drain_a(g):
        pltpu.make_async_copy(
            bhs[0].at[pl.ds(0, GS * 128)],
            vals_v.at[pl.ds(g * GS * 128, GS * 128)], sgs[0]).wait()

    for k in range(3):
        pltpu.sync_copy(facesT_hbm.at[k, sid], fidx_v)
        fire_a(0)

        def step_a(g, carry):
            fire_a(g + 1)
            drain_a(g)
            return carry

        lax.fori_loop(0, NG - 1, step_a, 0)
        drain_a(NG - 1)
        fstage = pltpu.make_async_copy(
            vals_v, shs[k].at[pl.ds(sid * SL, SL)], sgs[0])
        fstage.start()
        fstage.wait()
    plsc.subcore_barrier()

    def in_descs(c, q):
        row0 = wid * (PT // 128) + c * NR
        base = wid * PT + c * CH
        ds = [pltpu.make_async_copy(
            p2f_hbm.at[pl.ds(row0, NR)], idxs[q], sins[q])]
        for k in range(3):
            ds.append(pltpu.make_async_copy(
                bhs[k].at[pl.ds(base, CH)], bss[q][k], sins[q]))
        return ds

    def start_in(c, q):
        for d in in_descs(c, q):
            d.start()

    def wait_in(c, q):
        for d in in_descs(c, q):
            d.wait()

    def fire_g(q):
        for k in range(3):
            for j in range(NR):
                pltpu.make_async_copy(
                    shs[k].at[idxs[q].at[j]],
                    gss[q][k].at[pl.ds(j * 128, 128)], sgs[q]).start()

    def drain_g(q):
        for k in range(3):
            pltpu.make_async_copy(
                bhs[k].at[pl.ds(0, CH)], gss[q][k], sgs[q]).wait()

    def out_desc(c, q):
        base = wid * PT + c * CH
        return pltpu.make_async_copy(
            outs[q], out_hbm.at[pl.ds(base, CH)], sos[q])

    def compute(q):
        gs, bs, out_v = gss[q], bss[q], outs[q]

        def pix(i, carry2):
            for u in range(4):
                s = pl.ds(i * 4 * L + u * L, L)
                acc = gs[0][s] * bs[0][s]
                acc = acc + gs[1][s] * bs[1][s]
                acc = acc + gs[2][s] * bs[2][s]
                out_v[s] = acc.astype(jnp.int32)
            return carry2

        lax.fori_loop(0, CH // L // 4, pix, 0)

    def step(c, q, first):
        drain_g(q)
        wait_in(c + 2, (q + 2) % NB)
        fire_g((q + 2) % NB)
        if not first:
            out_desc(c - NB, q).wait()
        compute(q)
        out_desc(c, q).start()
        start_in(c + 3, (q + 3) % NB)

    start_in(0, 0)
    start_in(1, 1)
    wait_in(0, 0)
    fire_g(0)
    wait_in(1, 1)
    fire_g(1)
    start_in(2, 2)

    for c in range(NB):
        step(c, c % NB, True)

    def loop_body(t, carry):
        c0 = NB + t * NB
        for u in range(NB):
            step(c0 + u, u, False)
        return carry

    lax.fori_loop(0, (NC - 2 * NB) // NB, loop_body, 0)

    for c in range(NC - NB, NC):
        q = c % NB
        drain_g(q)
        if c + 2 < NC:
            wait_in(c + 2, (c + 2) % NB)
            fire_g((c + 2) % NB)
        out_desc(c - NB, q).wait()
        compute(q)
        out_desc(c, q).start()
        if c + 3 < NC:
            start_in(c + 3, (c + 3) % NB)

    for c in range(NC - NB, NC):
        out_desc(c, c % NB).wait()


def kernel(verts_features, faces, pix_to_face, bary_coords):
    N, V, C = verts_features.shape
    F = faces.shape[0]
    _, H, W, K = pix_to_face.shape
    P = N * H * W * K

    mesh = plsc.VectorSubcoreMesh(core_axis_name="c", subcore_axis_name="s")

    RO = -(-F // (NS * 128))
    Fp = NS * RO * 128
    NVp = NS * (-(-(N * V) // (NS * 128)) * 128)
    verts_flat = jnp.pad(verts_features.reshape(N * V), (0, NVp - N * V))
    facesT = faces.T
    facesT = jnp.pad(facesT, ((0, 0), (0, Fp - F)))
    facesT = facesT.reshape(3, NS, RO, 128)

    PT = P // NW
    CH = 1024
    p2f = pix_to_face.reshape(P // 128, 128)
    baryT = bary_coords.reshape(P, 3).T

    shade = pl.kernel(
        functools.partial(_shade_body, PT, CH, NVp, Fp),
        mesh=mesh,
        out_type=jax.ShapeDtypeStruct((P,), jnp.int32),
        scratch_types=(
            [pltpu.VMEM_SHARED((NVp,), jnp.float32)]
            + [pltpu.VMEM_SHARED((Fp,), jnp.float32)] * 3
            + [pltpu.VMEM((Fp // NS // 128, 128), jnp.int32),
               pltpu.VMEM((Fp // NS,), jnp.float32)]
            + [pltpu.VMEM((CH // 128, 128), jnp.int32)] * NB
            + [pltpu.VMEM((CH,), jnp.float32)] * (3 * NB)
            + [pltpu.VMEM((CH,), jnp.float32)] * (3 * NB)
            + [pltpu.VMEM((CH,), jnp.int32)] * NB
            + [pltpu.SemaphoreType.DMA] * (3 * NB)
        ),
    )
    out = shade(verts_flat, facesT, p2f, baryT[0], baryT[1], baryT[2])
    return out.reshape(N, H, W, K)


# speedup vs baseline: 1.7697x; 1.0022x over previous
import functools

import jax
import jax.numpy as jnp
from jax import lax
from jax.experimental import pallas as pl
from jax.experimental.pallas import tpu as pltpu
from jax.experimental.pallas import tpu_sc as plsc

NW = 32
NS = 16
L = 16
NB = 4
GS = 7


def _wid():
    return lax.axis_index("s") * 2 + lax.axis_index("c")


def _shade_body(PT, CH, NVp, Fp, *refs):
    verts_hbm, facesT_hbm, p2f_hbm = refs[0:3]
    bhs = refs[3:6]
    out_hbm = refs[6]
    rest = refs[7:]
    shv = rest[0]
    shs = rest[1:4]
    fidx_v, vals_v = rest[4:6]
    rest = rest[6:]
    idxs = rest[0:NB]
    bss = [rest[NB + q * 3: NB + q * 3 + 3] for q in range(NB)]
    gss = [rest[NB + 3 * NB + q * 3: NB + 3 * NB + q * 3 + 3] for q in range(NB)]
    outs = rest[NB + 6 * NB: NB + 6 * NB + NB]
    sins = rest[NB + 7 * NB: NB + 7 * NB + NB]
    sgs = rest[NB + 8 * NB: NB + 8 * NB + NB]
    sos = rest[NB + 9 * NB: NB + 9 * NB + NB]

    wid = _wid()
    sid = lax.axis_index("s")
    NR = CH // 128
    NC = PT // CH

    def in_descs(c, q):
        row0 = wid * (PT // 128) + c * NR
        base = wid * PT + c * CH
        ds = [pltpu.make_async_copy(
            p2f_hbm.at[pl.ds(row0, NR)], idxs[q], sins[q])]
        for k in range(3):
            ds.append(pltpu.make_async_copy(
                bhs[k].at[pl.ds(base, CH)], bss[q][k], sins[q]))
        return ds

    def start_in(c, q):
        for d in in_descs(c, q):
            d.start()

    def wait_in(c, q):
        for d in in_descs(c, q):
            d.wait()

    VSL = NVp // NS
    SL = Fp // NS
    RO = SL // 128
    NG = RO // GS

    start_in(0, 0)
    start_in(1, 1)
    start_in(2, 2)

    vstage = pltpu.make_async_copy(
        verts_hbm.at[pl.ds(sid * VSL, VSL)],
        shv.at[pl.ds(sid * VSL, VSL)], sgs[0])
    vstage.start()
    vstage.wait()
    plsc.subcore_barrier()

    def fire_a(g):
        for t in range(GS):
            r = g * GS + t
            pltpu.make_async_copy(
                shv.at[fidx_v.at[r]],
                vals_v.at[pl.ds(r * 128, 128)], sgs[0]).start()

    def drain_a(g):
        pltpu.make_async_copy(
            bhs[0].at[pl.ds(0, GS * 128)],
            vals_v.at[pl.ds(g * GS * 128, GS * 128)], sgs[0]).wait()

    for k in range(3):
        pltpu.sync_copy(facesT_hbm.at[k, sid], fidx_v)
        fire_a(0)

        def step_a(g, carry):
            fire_a(g + 1)
            drain_a(g)
            return carry

        lax.fori_loop(0, NG - 1, step_a, 0)
        drain_a(NG - 1)
        fstage = pltpu.make_async_copy(
            vals_v, shs[k].at[pl.ds(sid * SL, SL)], sgs[0])
        fstage.start()
        fstage.wait()
    plsc.subcore_barrier()

    def fire_g(q):
        for k in range(3):
            for j in range(NR):
                pltpu.make_async_copy(
                    shs[k].at[idxs[q].at[j]],
                    gss[q][k].at[pl.ds(j * 128, 128)], sgs[q]).start()

    def drain_g(q):
        for k in range(3):
            pltpu.make_async_copy(
                bhs[k].at[pl.ds(0, CH)], gss[q][k], sgs[q]).wait()

    def out_desc(c, q):
        base = wid * PT + c * CH
        return pltpu.make_async_copy(
            outs[q], out_hbm.at[pl.ds(base, CH)], sos[q])

    def compute(q):
        gs, bs, out_v = gss[q], bss[q], outs[q]

        def pix(i, carry2):
            for u in range(4):
                s = pl.ds(i * 4 * L + u * L, L)
                acc = gs[0][s] * bs[0][s]
                acc = acc + gs[1][s] * bs[1][s]
                acc = acc + gs[2][s] * bs[2][s]
                out_v[s] = acc.astype(jnp.int32)
            return carry2

        lax.fori_loop(0, CH // L // 4, pix, 0)

    def step(c, q, first):
        drain_g(q)
        wait_in(c + 2, (q + 2) % NB)
        fire_g((q + 2) % NB)
        if not first:
            out_desc(c - NB, q).wait()
        compute(q)
        out_desc(c, q).start()
        start_in(c + 3, (q + 3) % NB)

    wait_in(0, 0)
    fire_g(0)
    wait_in(1, 1)
    fire_g(1)

    for c in range(NB):
        step(c, c % NB, True)

    def loop_body(t, carry):
        c0 = NB + t * NB
        for u in range(NB):
            step(c0 + u, u, False)
        return carry

    lax.fori_loop(0, (NC - 2 * NB) // NB, loop_body, 0)

    for c in range(NC - NB, NC):
        q = c % NB
        drain_g(q)
        if c + 2 < NC:
            wait_in(c + 2, (c + 2) % NB)
            fire_g((c + 2) % NB)
        out_desc(c - NB, q).wait()
        compute(q)
        out_desc(c, q).start()
        if c + 3 < NC:
            start_in(c + 3, (c + 3) % NB)

    for c in range(NC - NB, NC):
        out_desc(c, c % NB).wait()


def kernel(verts_features, faces, pix_to_face, bary_coords):
    N, V, C = verts_features.shape
    F = faces.shape[0]
    _, H, W, K = pix_to_face.shape
    P = N * H * W * K

    mesh = plsc.VectorSubcoreMesh(core_axis_name="c", subcore_axis_name="s")

    RO = -(-F // (NS * 128))
    Fp = NS * RO * 128
    NVp = NS * (-(-(N * V) // (NS * 128)) * 128)
    verts_flat = jnp.pad(verts_features.reshape(N * V), (0, NVp - N * V))
    facesT = faces.T
    facesT = jnp.pad(facesT, ((0, 0), (0, Fp - F)))
    facesT = facesT.reshape(3, NS, RO, 128)

    PT = P // NW
    CH = 1024
    p2f = pix_to_face.reshape(P // 128, 128)
    baryT = bary_coords.reshape(P, 3).T

    shade = pl.kernel(
        functools.partial(_shade_body, PT, CH, NVp, Fp),
        mesh=mesh,
        out_type=jax.ShapeDtypeStruct((P,), jnp.int32),
        scratch_types=(
            [pltpu.VMEM_SHARED((NVp,), jnp.float32)]
            + [pltpu.VMEM_SHARED((Fp,), jnp.float32)] * 3
            + [pltpu.VMEM((Fp // NS // 128, 128), jnp.int32),
               pltpu.VMEM((Fp // NS,), jnp.float32)]
            + [pltpu.VMEM((CH // 128, 128), jnp.int32)] * NB
            + [pltpu.VMEM((CH,), jnp.float32)] * (3 * NB)
            + [pltpu.VMEM((CH,), jnp.float32)] * (3 * NB)
            + [pltpu.VMEM((CH,), jnp.int32)] * NB
            + [pltpu.SemaphoreType.DMA] * (3 * NB)
        ),
    )
    out = shade(verts_flat, facesT, p2f, baryT[0], baryT[1], baryT[2])
    return out.reshape(N, H, W, K)
